# Initial kernel scaffold; baseline (speedup 1.0000x reference)
import jax
import jax.numpy as jnp
from jax.experimental import pallas as pl


def kernel(q, k, v, numeric_embedding_manager, attention_mask):
    raise NotImplementedError("write your pallas kernel here")


# speedup vs baseline: 3.5577x; 3.5577x over previous
import functools
import math

import jax
import jax.numpy as jnp
from jax import lax
from jax.experimental import pallas as pl
from jax.experimental.pallas import tpu as pltpu
from jax.experimental.pallas import tpu_sc as plsc


def _sc_row_gather(table, idx):
    R, D = table.shape
    info = plsc.get_sparse_core_info()
    nw = info.num_cores * info.num_subcores
    b_per_w = R // nw

    mesh = plsc.VectorSubcoreMesh(core_axis_name="c", subcore_axis_name="s")

    @functools.partial(
        pl.kernel,
        mesh=mesh,
        out_type=jax.ShapeDtypeStruct((R, D), jnp.float32),
        scratch_types=[
            pltpu.VMEM((b_per_w,), jnp.int32),
            pltpu.VMEM((b_per_w, D), jnp.float32),
            pltpu.SemaphoreType.DMA,
        ],
    )
    def gather_kernel(table_hbm, idx_hbm, out_hbm, idx_v, rows_v, sem):
        wid = lax.axis_index("s") * info.num_cores + lax.axis_index("c")
        base = wid * b_per_w
        pltpu.sync_copy(idx_hbm.at[pl.ds(base, b_per_w)], idx_v)
        pltpu.async_copy(table_hbm.at[idx_v], rows_v, sem).wait()
        pltpu.sync_copy(rows_v, out_hbm.at[pl.ds(base, b_per_w)])

    return gather_kernel(table, idx)


def _attn_body(q_ref, k_ref, gk_ref, v_ref, gm_ref, vc_ref, rm_ref,
               probs_ref, ctx_ref, *, inv_scale):
    qb = q_ref[0]
    kk = k_ref[0]
    gkk = gk_ref[0]
    vv = v_ref[0]
    gm = gm_ref[...]
    vc = vc_ref[...]
    rm = rm_ref[...]

    dims = (((1,), (1,)), ((), ()))
    s_glob = lax.dot_general(qb, gkk, dims,
                             preferred_element_type=jnp.float32) * inv_scale
    s_glob = jnp.where(vc > 0.0, s_glob, -jnp.inf)
    m_gl = jnp.max(s_glob, axis=1, keepdims=True)
    e_gl = jnp.exp(s_glob - m_gl)
    p_out = e_gl * (1.0 / jnp.sum(e_gl, axis=1, keepdims=True))
    probs_ref[0] = p_out

    s_full = lax.dot_general(qb, kk, dims,
                             preferred_element_type=jnp.float32) * inv_scale
    e = jnp.exp(s_full - m_gl)
    den_all = jnp.sum(e, axis=1, keepdims=True)
    eg = e * gm
    den_g = jnp.sum(eg, axis=1, keepdims=True)
    p_ctx = jnp.where(rm > 0.0, e * (1.0 / den_all), eg * (1.0 / den_g))
    ctx_ref[0] = lax.dot_general(p_ctx, vv, (((1,), (0,)), ((), ())),
                                 preferred_element_type=jnp.float32)


def _attention(q2, k2, gk, v2, gmask_col, valid_col, rmask, *, bq):
    BH, S, D = q2.shape
    grid = (BH, S // bq)
    row_block = pl.BlockSpec((1, bq, D), lambda h, i: (h, i, 0))
    full_block = pl.BlockSpec((1, S, D), lambda h, i: (h, 0, 0))
    col_block = pl.BlockSpec((1, S), lambda h, i: (0, 0))
    probs_spec = pl.BlockSpec((1, bq, S), lambda h, i: (h, i, 0))
    rm_spec = pl.BlockSpec((bq, 1), lambda h, i: (i, 0))
    return pl.pallas_call(
        functools.partial(_attn_body, inv_scale=1.0 / math.sqrt(D)),
        grid=grid,
        in_specs=[row_block, full_block, full_block, full_block,
                  col_block, col_block, rm_spec],
        out_specs=[probs_spec, row_block],
        out_shape=[
            jax.ShapeDtypeStruct((BH, S, S), jnp.float32),
            jax.ShapeDtypeStruct((BH, S, D), jnp.float32),
        ],
        compiler_params=pltpu.CompilerParams(
            dimension_semantics=("arbitrary", "arbitrary"),
        ),
    )(q2, k2, gk, v2, gmask_col, valid_col, rmask)


def kernel(q, k, v, numeric_embedding_manager, attention_mask):
    B, H, S, D = q.shape
    BH = B * H
    q2 = q.reshape(BH, S, D)
    k2 = k.reshape(BH, S, D)
    v2 = v.reshape(BH, S, D)

    is_g = (attention_mask > 0)
    order = jnp.argsort(jnp.logical_not(is_g).astype(jnp.int32),
                        axis=1, stable=True).astype(jnp.int32)
    n_glob = is_g.sum(axis=1).astype(jnp.int32)

    kt = k2.transpose(1, 0, 2).reshape(S, BH * D)
    gk_t = _sc_row_gather(kt, order.reshape(S))
    gk = gk_t.reshape(S, BH, D).transpose(1, 0, 2)

    pos = jnp.arange(S, dtype=jnp.int32)
    valid_b = (pos[None, :] < n_glob[:, None]).astype(jnp.float32)
    gmask_b = is_g.astype(jnp.float32)
    gmask_col = gmask_b[:1]
    valid_col = valid_b[:1]
    rmask = gmask_b[0][:, None]

    probs, ctx = _attention(q2, k2, gk, v2, gmask_col, valid_col, rmask,
                            bq=256)
    return ctx.reshape(B, H, S, D), probs.reshape(B, H, S, S)


# speedup vs baseline: 3.7756x; 1.0613x over previous
import functools
import math

import jax
import jax.numpy as jnp
from jax import lax
from jax.experimental import pallas as pl
from jax.experimental.pallas import tpu as pltpu
from jax.experimental.pallas import tpu_sc as plsc


def _sc_row_gather(table, idx):
    R, D = table.shape
    info = plsc.get_sparse_core_info()
    nw = info.num_cores * info.num_subcores
    b_per_w = R // nw

    mesh = plsc.VectorSubcoreMesh(core_axis_name="c", subcore_axis_name="s")

    @functools.partial(
        pl.kernel,
        mesh=mesh,
        out_type=jax.ShapeDtypeStruct((R, D), jnp.float32),
        scratch_types=[
            pltpu.VMEM((b_per_w,), jnp.int32),
            pltpu.VMEM((b_per_w, D), jnp.float32),
            pltpu.SemaphoreType.DMA,
        ],
    )
    def gather_kernel(table_hbm, idx_hbm, out_hbm, idx_v, rows_v, sem):
        wid = lax.axis_index("s") * info.num_cores + lax.axis_index("c")
        base = wid * b_per_w
        pltpu.sync_copy(idx_hbm.at[pl.ds(base, b_per_w)], idx_v)
        pltpu.async_copy(table_hbm.at[idx_v], rows_v, sem).wait()
        pltpu.sync_copy(rows_v, out_hbm.at[pl.ds(base, b_per_w)])

    return gather_kernel(table, idx)


def _attn_body(q_ref, k_ref, gk_ref, v_ref, gm_ref, vc_ref, rm_ref,
               probs_ref, ctx_ref, *, inv_scale):
    qb = q_ref[0]
    kk = k_ref[0]
    gkk = gk_ref[0]
    vv = v_ref[0]
    gm = gm_ref[...]
    vc = vc_ref[...]
    rm = rm_ref[...]

    qb16 = qb.astype(jnp.bfloat16)
    dims = (((1,), (1,)), ((), ()))
    s_glob = lax.dot_general(qb16, gkk.astype(jnp.bfloat16), dims,
                             preferred_element_type=jnp.float32) * inv_scale
    s_glob = jnp.where(vc > 0.0, s_glob, -jnp.inf)
    m_gl = jnp.max(s_glob, axis=1, keepdims=True)
    e_gl = jnp.exp(s_glob - m_gl)
    p_out = e_gl * (1.0 / jnp.sum(e_gl, axis=1, keepdims=True))
    probs_ref[0] = p_out

    s_full = lax.dot_general(qb16, kk.astype(jnp.bfloat16), dims,
                             preferred_element_type=jnp.float32) * inv_scale
    e = jnp.exp(s_full - m_gl)
    den_all = jnp.sum(e, axis=1, keepdims=True)
    eg = e * gm
    den_g = jnp.sum(eg, axis=1, keepdims=True)
    p_ctx = jnp.where(rm > 0.0, e * (1.0 / den_all), eg * (1.0 / den_g))
    ctx_ref[0] = lax.dot_general(p_ctx.astype(jnp.bfloat16),
                                 vv.astype(jnp.bfloat16),
                                 (((1,), (0,)), ((), ())),
                                 preferred_element_type=jnp.float32)


def _attention(q2, k2, gk, v2, gmask_col, valid_col, rmask, *, bq):
    BH, S, D = q2.shape
    grid = (BH, S // bq)
    row_block = pl.BlockSpec((1, bq, D), lambda h, i: (h, i, 0))
    full_block = pl.BlockSpec((1, S, D), lambda h, i: (h, 0, 0))
    col_block = pl.BlockSpec((1, S), lambda h, i: (0, 0))
    probs_spec = pl.BlockSpec((1, bq, S), lambda h, i: (h, i, 0))
    rm_spec = pl.BlockSpec((bq, 1), lambda h, i: (i, 0))
    return pl.pallas_call(
        functools.partial(_attn_body, inv_scale=1.0 / math.sqrt(D)),
        grid=grid,
        in_specs=[row_block, full_block, full_block, full_block,
                  col_block, col_block, rm_spec],
        out_specs=[probs_spec, row_block],
        out_shape=[
            jax.ShapeDtypeStruct((BH, S, S), jnp.float32),
            jax.ShapeDtypeStruct((BH, S, D), jnp.float32),
        ],
        compiler_params=pltpu.CompilerParams(
            dimension_semantics=("arbitrary", "arbitrary"),
        ),
    )(q2, k2, gk, v2, gmask_col, valid_col, rmask)


def kernel(q, k, v, numeric_embedding_manager, attention_mask):
    B, H, S, D = q.shape
    BH = B * H
    q2 = q.reshape(BH, S, D)
    k2 = k.reshape(BH, S, D)
    v2 = v.reshape(BH, S, D)

    is_g = (attention_mask > 0)
    order = jnp.argsort(jnp.logical_not(is_g).astype(jnp.int32),
                        axis=1, stable=True).astype(jnp.int32)
    n_glob = is_g.sum(axis=1).astype(jnp.int32)

    kt = k2.transpose(1, 0, 2).reshape(S, BH * D)
    gk_t = _sc_row_gather(kt, order.reshape(S))
    gk = gk_t.reshape(S, BH, D).transpose(1, 0, 2)

    pos = jnp.arange(S, dtype=jnp.int32)
    valid_b = (pos[None, :] < n_glob[:, None]).astype(jnp.float32)
    gmask_b = is_g.astype(jnp.float32)
    gmask_col = gmask_b[:1]
    valid_col = valid_b[:1]
    rmask = gmask_b[0][:, None]

    probs, ctx = _attention(q2, k2, gk, v2, gmask_col, valid_col, rmask,
                            bq=256)
    return ctx.reshape(B, H, S, D), probs.reshape(B, H, S, S)
